# Initial kernel scaffold; baseline (speedup 1.0000x reference)
import jax
import jax.numpy as jnp
from jax.experimental import pallas as pl


def kernel(x, edge_index, edge_attr, node_W, node_b, edge_W, edge_b, t0, W1_0, b1_0, g1_0, be1_0, W2_0, b2_0, ln_g0, ln_b0, t1, W1_1, b1_1, g1_1, be1_1, W2_1, b2_1, ln_g1, ln_b1, out_W, out_b):
    raise NotImplementedError("write your pallas kernel here")


# speedup vs baseline: 2.2274x; 2.2274x over previous
import functools

import jax
import jax.numpy as jnp
from jax import lax
from jax.experimental import pallas as pl
from jax.experimental.pallas import tpu as pltpu
from jax.experimental.pallas import tpu_sc as plsc

N = 10000
E = 320000
D_EDGE = 16
H = 128
HH = 64

NUM_TILES = 16
CHUNK = 128
CHUNKS_PER_TILE = 157
E_PAD = NUM_TILES * CHUNKS_PER_TILE * CHUNK
N_ACC = 10240
ROWS_PER_TILE = N_ACC // NUM_TILES
ROW_CHUNK = 128

def _sc_aggregate_body(htab, e2, src, dst, tvec, agg, acc, sidx, didx, g_v,
                       e_v, o_v, tv, sem_g):
    c = lax.axis_index("c")
    s = lax.axis_index("s")

    pltpu.sync_copy(tvec, tv)

    def _zrow(i, _):
        for k in range(8):
            o_v[i, pl.ds(16 * k, 16)] = jnp.zeros((16,), jnp.float32)
        return 0

    lax.fori_loop(0, CHUNK, _zrow, 0)

    zbase = s * ROWS_PER_TILE

    def _zcopy(j, _):
        z0 = pl.multiple_of(zbase + j * ROW_CHUNK, 8)
        pltpu.sync_copy(o_v, acc.at[pl.ds(z0, ROW_CHUNK)])
        return 0

    lax.fori_loop(0, ROWS_PER_TILE // ROW_CHUNK, _zcopy, 0)

    plsc.subcore_barrier()

    ept = E_PAD // NUM_TILES
    wbase = s * ept
    e_base = c * (E_PAD // 2) + wbase // 2
    cb = c * HH
    tvv = tv[...]

    def _chunk(ci, _):
        base = pl.multiple_of(wbase + ci * CHUNK, 8)
        eb = pl.multiple_of(e_base + ci * (CHUNK // 2), 8)
        pltpu.sync_copy(src.at[pl.ds(base, CHUNK)], sidx.at[0])
        pltpu.sync_copy(dst.at[pl.ds(base, CHUNK)], didx.at[0])
        pltpu.async_copy(htab.at[sidx.at[0]], g_v, sem_g).wait()
        pltpu.sync_copy(e2.at[pl.ds(eb, CHUNK // 2)], e_v)

        def _edge(rr, _):
            for half in range(2):
                i = 2 * rr + half
                for k in range(4):
                    g = g_v[i, pl.ds(cb + 16 * k, 16)]
                    ee = e_v[rr, pl.ds(HH * half + 16 * k, 16)]
                    m = jnp.maximum(g + ee, 0.0) + 1e-7
                    w = jnp.exp(m * tvv)
                    o_v[i, pl.ds(16 * k, 16)] = m * w
                    o_v[i, pl.ds(HH + 16 * k, 16)] = w
            return 0

        lax.fori_loop(0, CHUNK // 2, _edge, 0)
        pltpu.sync_copy(o_v, acc.at[didx.at[0]], add=True)
        return 0

    lax.fori_loop(0, CHUNKS_PER_TILE, _chunk, 0)

    plsc.subcore_barrier()

    rbase = s * ROWS_PER_TILE

    def _out(j, _):
        r0 = pl.multiple_of(rbase + j * ROW_CHUNK, 8)
        pltpu.sync_copy(acc.at[pl.ds(r0, ROW_CHUNK)], o_v)

        def _row(r, _):
            for k in range(4):
                num = o_v[r, pl.ds(16 * k, 16)]
                den = o_v[r, pl.ds(HH + 16 * k, 16)]
                g_v[r, pl.ds(cb + 16 * k, 16)] = num / (den + 1e-16)
            return 0

        lax.fori_loop(0, ROW_CHUNK, _row, 0)
        pltpu.sync_copy(g_v, agg.at[c, pl.ds(r0, ROW_CHUNK)])
        return 0

    lax.fori_loop(0, ROWS_PER_TILE // ROW_CHUNK, _out, 0)


@functools.cache
def _sc_aggregate():
    mesh = plsc.VectorSubcoreMesh(
        core_axis_name="c", subcore_axis_name="s", num_cores=2, num_subcores=16
    )
    return pl.kernel(
        _sc_aggregate_body,
        out_type=jax.ShapeDtypeStruct((2, N_ACC, H), jnp.float32),
        mesh=mesh,
        scratch_types=[
            pltpu.VMEM_SHARED((N_ACC, H), jnp.float32),
            pltpu.VMEM((2, CHUNK), jnp.int32),
            pltpu.VMEM((2, CHUNK), jnp.int32),
            pltpu.VMEM((CHUNK, H), jnp.float32),
            pltpu.VMEM((CHUNK // 2, H), jnp.float32),
            pltpu.VMEM((CHUNK, H), jnp.float32),
            pltpu.VMEM((16,), jnp.float32),
            pltpu.SemaphoreType.DMA,
        ],
    )


_NBLK = 1000
_EBLK = 2048


def _ln(z, g, b):
    mu = jnp.mean(z, axis=-1, keepdims=True)
    var = jnp.mean((z - mu) ** 2, axis=-1, keepdims=True)
    return (z - mu) / jnp.sqrt(var + 1e-5) * g + b


def _enc_node_body(x_ref, w_ref, b_ref, out_ref):
    h = jnp.dot(x_ref[...], w_ref[...], preferred_element_type=jnp.float32)
    out_ref[...] = h + b_ref[...]


def _enc_edge_body(a_ref, wd_ref, bd_ref, out_ref):
    a = a_ref[...]
    for c in range(2):
        z = jnp.dot(a, wd_ref[c], preferred_element_type=jnp.float32)
        out_ref[c] = jax.nn.sigmoid(z + bd_ref[c])


def _update1_body(agg_ref, h_ref, w1_ref, b1_ref, g1_ref, be1_ref, w2_ref,
                  b2_ref, lng_ref, lnb_ref, h1_ref, r_ref):
    agg = jnp.concatenate([agg_ref[0][:, :HH], agg_ref[1][:, HH:]], axis=1)
    u = h_ref[...] + agg
    z = jnp.dot(u, w1_ref[...], preferred_element_type=jnp.float32) + b1_ref[...]
    z = jax.nn.relu(_ln(z, g1_ref[...], be1_ref[...]))
    h1 = jnp.dot(z, w2_ref[...], preferred_element_type=jnp.float32) + b2_ref[...]
    h1_ref[...] = h1
    r_ref[...] = jax.nn.relu(_ln(h1, lng_ref[...], lnb_ref[...]))


def _update2_body(agg_ref, r_ref, h1_ref, w1_ref, b1_ref, g1_ref, be1_ref,
                  w2_ref, b2_ref, lng_ref, lnb_ref, ow_ref, ob_ref, out_ref):
    agg = jnp.concatenate([agg_ref[0][:, :HH], agg_ref[1][:, HH:]], axis=1)
    u = r_ref[...] + agg
    z = jnp.dot(u, w1_ref[...], preferred_element_type=jnp.float32) + b1_ref[...]
    z = jax.nn.relu(_ln(z, g1_ref[...], be1_ref[...]))
    h2 = h1_ref[...] + jnp.dot(z, w2_ref[...], preferred_element_type=jnp.float32) + b2_ref[...]
    f = jax.nn.relu(_ln(h2, lng_ref[...], lnb_ref[...]))
    out_ref[...] = jnp.dot(f, ow_ref[...], preferred_element_type=jnp.float32) + ob_ref[...]


def _full(shape):
    return pl.BlockSpec(shape, lambda i: tuple(0 for _ in shape))


def _row(v):
    return v.reshape(1, -1)


def kernel(x, edge_index, edge_attr, node_W, node_b, edge_W, edge_b, t0, W1_0,
           b1_0, g1_0, be1_0, W2_0, b2_0, ln_g0, ln_b0, t1, W1_1, b1_1, g1_1,
           be1_1, W2_1, b2_1, ln_g1, ln_b1, out_W, out_b):
    pad = E_PAD - E
    src_p = jnp.concatenate([edge_index[0], jnp.zeros((pad,), jnp.int32)])
    dst_p = jnp.concatenate([edge_index[1], jnp.full((pad,), N, jnp.int32)])
    ea_p = jnp.concatenate([edge_attr, jnp.zeros((pad, D_EDGE), jnp.float32)])

    h = pl.pallas_call(
        _enc_node_body,
        grid=(N // _NBLK,),
        in_specs=[
            pl.BlockSpec((_NBLK, H), lambda i: (i, 0)),
            _full((H, H)),
            _full((1, H)),
        ],
        out_specs=pl.BlockSpec((_NBLK, H), lambda i: (i, 0)),
        out_shape=jax.ShapeDtypeStruct((N, H), jnp.float32),
    )(x, node_W, _row(node_b))

    ea2 = ea_p.reshape(E_PAD // 2, 2 * D_EDGE)
    z16 = jnp.zeros((D_EDGE, HH), jnp.float32)
    wd = jnp.stack([
        jnp.concatenate([
            jnp.concatenate([edge_W[:, c * HH:(c + 1) * HH], z16], axis=1),
            jnp.concatenate([z16, edge_W[:, c * HH:(c + 1) * HH]], axis=1),
        ], axis=0) for c in range(2)
    ])
    bd = jnp.stack([jnp.tile(edge_b[c * HH:(c + 1) * HH], 2) for c in range(2)])

    e2 = pl.pallas_call(
        _enc_edge_body,
        grid=(E_PAD // _EBLK,),
        in_specs=[
            pl.BlockSpec((_EBLK // 2, 2 * D_EDGE), lambda i: (i, 0)),
            _full((2, 2 * D_EDGE, H)),
            _full((2, H)),
        ],
        out_specs=pl.BlockSpec((2, _EBLK // 2, H), lambda i: (0, i, 0)),
        out_shape=jax.ShapeDtypeStruct((2, E_PAD // 2, H), jnp.float32),
    )(ea2, wd, bd)

    e2_flat = e2.reshape(E_PAD, H)

    agg0 = _sc_aggregate()(h, e2_flat, src_p, dst_p,
                           jnp.full((16,), t0, jnp.float32))

    h1, r = pl.pallas_call(
        _update1_body,
        grid=(N // _NBLK,),
        in_specs=[
            pl.BlockSpec((2, _NBLK, H), lambda i: (0, i, 0)),
            pl.BlockSpec((_NBLK, H), lambda i: (i, 0)),
            _full((H, 2 * H)),
            _full((1, 2 * H)),
            _full((1, 2 * H)),
            _full((1, 2 * H)),
            _full((2 * H, H)),
            _full((1, H)),
            _full((1, H)),
            _full((1, H)),
        ],
        out_specs=[
            pl.BlockSpec((_NBLK, H), lambda i: (i, 0)),
            pl.BlockSpec((_NBLK, H), lambda i: (i, 0)),
        ],
        out_shape=[
            jax.ShapeDtypeStruct((N, H), jnp.float32),
            jax.ShapeDtypeStruct((N, H), jnp.float32),
        ],
    )(agg0, h, W1_0, _row(b1_0), _row(g1_0), _row(be1_0), W2_0, _row(b2_0),
      _row(ln_g1), _row(ln_b1))

    agg1 = _sc_aggregate()(r, e2_flat, src_p, dst_p,
                           jnp.full((16,), t1, jnp.float32))

    out = pl.pallas_call(
        _update2_body,
        grid=(N // _NBLK,),
        in_specs=[
            pl.BlockSpec((2, _NBLK, H), lambda i: (0, i, 0)),
            pl.BlockSpec((_NBLK, H), lambda i: (i, 0)),
            pl.BlockSpec((_NBLK, H), lambda i: (i, 0)),
            _full((H, 2 * H)),
            _full((1, 2 * H)),
            _full((1, 2 * H)),
            _full((1, 2 * H)),
            _full((2 * H, H)),
            _full((1, H)),
            _full((1, H)),
            _full((1, H)),
            _full((H, H)),
            _full((1, H)),
        ],
        out_specs=pl.BlockSpec((_NBLK, H), lambda i: (i, 0)),
        out_shape=jax.ShapeDtypeStruct((N, H), jnp.float32),
    )(agg1, r, h1, W1_1, _row(b1_1), _row(g1_1), _row(be1_1), W2_1,
      _row(b2_1), _row(ln_g0), _row(ln_b0), out_W, _row(out_b))

    return out


# speedup vs baseline: 2.7224x; 1.2222x over previous
import functools

import jax
import jax.numpy as jnp
from jax import lax
from jax.experimental import pallas as pl
from jax.experimental.pallas import tpu as pltpu
from jax.experimental.pallas import tpu_sc as plsc

N = 10000
E = 320000
D_EDGE = 16
H = 128
HH = 64

NUM_TILES = 16
CHUNK = 32
CH2 = CHUNK // 2
NCH = 640
IBLK = 32
NIB = NCH // IBLK
E_PAD = NUM_TILES * NCH * CHUNK
N_ACC = 10240
ROWS_PER_TILE = N_ACC // NUM_TILES

def _sc_aggregate_body(htab, e2, src2, dst2, tvec, agg, acc, sidx, didx, g_v,
                       e_v, o_v, tv, gsem, esem, ssem):
    c = lax.axis_index("c")
    s = lax.axis_index("s")

    pltpu.sync_copy(tvec, tv)

    def _zrow(i, _):
        for b in range(2):
            for k in range(8):
                o_v[b, i, pl.ds(16 * k, 16)] = jnp.zeros((16,), jnp.float32)
        return 0

    lax.fori_loop(0, CHUNK, _zrow, 0)

    zbase = s * ROWS_PER_TILE

    def _zcopy(j, _):
        z0 = pl.multiple_of(zbase + j * CHUNK, 8)
        pltpu.sync_copy(o_v.at[0], acc.at[pl.ds(z0, CHUNK)])
        return 0

    lax.fori_loop(0, ROWS_PER_TILE // CHUNK, _zcopy, 0)

    plsc.subcore_barrier()

    wbase = s * NCH * CHUNK
    irow = s * NCH
    e_base = c * (E_PAD // 2) + wbase // 2
    cb = c * HH
    tvv = tv[...]

    def _fire_in(ci, b):
        q = lax.rem(ci, IBLK)
        s2 = lax.rem(ci // IBLK, 2)
        pltpu.async_copy(htab.at[sidx.at[s2, q]], g_v.at[b], gsem)
        eb = pl.multiple_of(e_base + ci * CH2, 8)
        pltpu.async_copy(e2.at[pl.ds(eb, CH2)], e_v.at[b], esem)

    pltpu.sync_copy(src2.at[pl.ds(pl.multiple_of(irow, 8), IBLK)], sidx.at[0])
    pltpu.sync_copy(dst2.at[pl.ds(pl.multiple_of(irow, 8), IBLK)], didx)
    for b in range(2):
        _fire_in(b, b)

    def _blk(bi, _):
        @pl.when(bi > 0)
        def _():
            for _i in range(2):
                pltpu.make_async_copy(o_v.at[0], acc.at[didx.at[0]], ssem).wait()
            r0 = pl.multiple_of(irow + bi * IBLK, 8)
            pltpu.sync_copy(dst2.at[pl.ds(r0, IBLK)], didx)

        @pl.when(bi < NIB - 1)
        def _():
            r1 = pl.multiple_of(irow + (bi + 1) * IBLK, 8)
            pltpu.sync_copy(src2.at[pl.ds(r1, IBLK)], sidx.at[lax.rem(bi + 1, 2)])

        def _pair(cp, _):
            for b in range(2):
                lq = 2 * cp + b
                ci = bi * IBLK + lq
                pltpu.make_async_copy(htab.at[sidx.at[0, 0]], g_v.at[b], gsem).wait()
                pltpu.make_async_copy(e2.at[pl.ds(0, CH2)], e_v.at[b], esem).wait()

                @pl.when(cp >= 1)
                def _():
                    pltpu.make_async_copy(o_v.at[0], acc.at[didx.at[0]], ssem).wait()

                def _edge(rr, _):
                    for half in range(2):
                        i = 2 * rr + half
                        for k in range(4):
                            g = g_v[b, i, pl.ds(cb + 16 * k, 16)]
                            ee = e_v[b, rr, pl.ds(HH * half + 16 * k, 16)]
                            m = jnp.maximum(g + ee, 0.0) + 1e-7
                            w = jnp.exp(m * tvv)
                            o_v[b, i, pl.ds(16 * k, 16)] = m * w
                            o_v[b, i, pl.ds(HH + 16 * k, 16)] = w
                    return 0

                lax.fori_loop(0, CH2, _edge, 0)
                pltpu.async_copy(o_v.at[b], acc.at[didx.at[lq]], ssem, add=True)

                @pl.when(ci + 2 < NCH)
                def _():
                    _fire_in(ci + 2, b)
            return 0

        lax.fori_loop(0, IBLK // 2, _pair, 0)
        return 0

    lax.fori_loop(0, NIB, _blk, 0)

    for _i in range(2):
        pltpu.make_async_copy(o_v.at[0], acc.at[didx.at[0]], ssem).wait()

    plsc.subcore_barrier()

    rbase = s * ROWS_PER_TILE

    def _out(j, _):
        r0 = pl.multiple_of(rbase + j * CHUNK, 8)
        pltpu.sync_copy(acc.at[pl.ds(r0, CHUNK)], o_v.at[0])

        def _row(r, _):
            for k in range(4):
                num = o_v[0, r, pl.ds(16 * k, 16)]
                den = o_v[0, r, pl.ds(HH + 16 * k, 16)]
                g_v[0, r, pl.ds(cb + 16 * k, 16)] = num / (den + 1e-16)
            return 0

        lax.fori_loop(0, CHUNK, _row, 0)
        pltpu.sync_copy(g_v.at[0], agg.at[c, pl.ds(r0, CHUNK)])
        return 0

    lax.fori_loop(0, ROWS_PER_TILE // CHUNK, _out, 0)


@functools.cache
def _sc_aggregate():
    mesh = plsc.VectorSubcoreMesh(
        core_axis_name="c", subcore_axis_name="s", num_cores=2, num_subcores=16
    )
    return pl.kernel(
        _sc_aggregate_body,
        out_type=jax.ShapeDtypeStruct((2, N_ACC, H), jnp.float32),
        mesh=mesh,
        scratch_types=[
            pltpu.VMEM_SHARED((N_ACC, H), jnp.float32),
            pltpu.VMEM((2, IBLK, CHUNK), jnp.int32),
            pltpu.VMEM((IBLK, CHUNK), jnp.int32),
            pltpu.VMEM((2, CHUNK, H), jnp.float32),
            pltpu.VMEM((2, CH2, H), jnp.float32),
            pltpu.VMEM((2, CHUNK, H), jnp.float32),
            pltpu.VMEM((16,), jnp.float32),
            pltpu.SemaphoreType.DMA,
            pltpu.SemaphoreType.DMA,
            pltpu.SemaphoreType.DMA,
        ],
    )


_NBLK = 1000
_EBLK = 2048


def _ln(z, g, b):
    mu = jnp.mean(z, axis=-1, keepdims=True)
    var = jnp.mean((z - mu) ** 2, axis=-1, keepdims=True)
    return (z - mu) / jnp.sqrt(var + 1e-5) * g + b


def _enc_node_body(x_ref, w_ref, b_ref, out_ref):
    h = jnp.dot(x_ref[...], w_ref[...], preferred_element_type=jnp.float32)
    out_ref[...] = h + b_ref[...]


def _enc_edge_body(a_ref, wd_ref, bd_ref, out_ref):
    a = a_ref[...]
    for c in range(2):
        z = jnp.dot(a, wd_ref[c], preferred_element_type=jnp.float32)
        out_ref[c] = jax.nn.sigmoid(z + bd_ref[c])


def _update1_body(agg_ref, h_ref, w1_ref, b1_ref, g1_ref, be1_ref, w2_ref,
                  b2_ref, lng_ref, lnb_ref, h1_ref, r_ref):
    agg = jnp.concatenate([agg_ref[0][:, :HH], agg_ref[1][:, HH:]], axis=1)
    u = h_ref[...] + agg
    z = jnp.dot(u, w1_ref[...], preferred_element_type=jnp.float32) + b1_ref[...]
    z = jax.nn.relu(_ln(z, g1_ref[...], be1_ref[...]))
    h1 = jnp.dot(z, w2_ref[...], preferred_element_type=jnp.float32) + b2_ref[...]
    h1_ref[...] = h1
    r_ref[...] = jax.nn.relu(_ln(h1, lng_ref[...], lnb_ref[...]))


def _update2_body(agg_ref, r_ref, h1_ref, w1_ref, b1_ref, g1_ref, be1_ref,
                  w2_ref, b2_ref, lng_ref, lnb_ref, ow_ref, ob_ref, out_ref):
    agg = jnp.concatenate([agg_ref[0][:, :HH], agg_ref[1][:, HH:]], axis=1)
    u = r_ref[...] + agg
    z = jnp.dot(u, w1_ref[...], preferred_element_type=jnp.float32) + b1_ref[...]
    z = jax.nn.relu(_ln(z, g1_ref[...], be1_ref[...]))
    h2 = h1_ref[...] + jnp.dot(z, w2_ref[...], preferred_element_type=jnp.float32) + b2_ref[...]
    f = jax.nn.relu(_ln(h2, lng_ref[...], lnb_ref[...]))
    out_ref[...] = jnp.dot(f, ow_ref[...], preferred_element_type=jnp.float32) + ob_ref[...]


def _full(shape):
    return pl.BlockSpec(shape, lambda i: tuple(0 for _ in shape))


def _row(v):
    return v.reshape(1, -1)


def kernel(x, edge_index, edge_attr, node_W, node_b, edge_W, edge_b, t0, W1_0,
           b1_0, g1_0, be1_0, W2_0, b2_0, ln_g0, ln_b0, t1, W1_1, b1_1, g1_1,
           be1_1, W2_1, b2_1, ln_g1, ln_b1, out_W, out_b):
    pad = E_PAD - E
    src_p = jnp.concatenate([edge_index[0], jnp.zeros((pad,), jnp.int32)])
    dst_p = jnp.concatenate([edge_index[1], jnp.full((pad,), N, jnp.int32)])
    ea_p = jnp.concatenate([edge_attr, jnp.zeros((pad, D_EDGE), jnp.float32)])

    h = pl.pallas_call(
        _enc_node_body,
        grid=(N // _NBLK,),
        in_specs=[
            pl.BlockSpec((_NBLK, H), lambda i: (i, 0)),
            _full((H, H)),
            _full((1, H)),
        ],
        out_specs=pl.BlockSpec((_NBLK, H), lambda i: (i, 0)),
        out_shape=jax.ShapeDtypeStruct((N, H), jnp.float32),
    )(x, node_W, _row(node_b))

    ea2 = ea_p.reshape(E_PAD // 2, 2 * D_EDGE)
    z16 = jnp.zeros((D_EDGE, HH), jnp.float32)
    wd = jnp.stack([
        jnp.concatenate([
            jnp.concatenate([edge_W[:, c * HH:(c + 1) * HH], z16], axis=1),
            jnp.concatenate([z16, edge_W[:, c * HH:(c + 1) * HH]], axis=1),
        ], axis=0) for c in range(2)
    ])
    bd = jnp.stack([jnp.tile(edge_b[c * HH:(c + 1) * HH], 2) for c in range(2)])

    e2 = pl.pallas_call(
        _enc_edge_body,
        grid=(E_PAD // _EBLK,),
        in_specs=[
            pl.BlockSpec((_EBLK // 2, 2 * D_EDGE), lambda i: (i, 0)),
            _full((2, 2 * D_EDGE, H)),
            _full((2, H)),
        ],
        out_specs=pl.BlockSpec((2, _EBLK // 2, H), lambda i: (0, i, 0)),
        out_shape=jax.ShapeDtypeStruct((2, E_PAD // 2, H), jnp.float32),
    )(ea2, wd, bd)

    e2_flat = e2.reshape(E_PAD, H)

    src2 = src_p.reshape(E_PAD // CHUNK, CHUNK)
    dst2 = dst_p.reshape(E_PAD // CHUNK, CHUNK)

    agg0 = _sc_aggregate()(h, e2_flat, src2, dst2,
                           jnp.full((16,), t0, jnp.float32))

    h1, r = pl.pallas_call(
        _update1_body,
        grid=(N // _NBLK,),
        in_specs=[
            pl.BlockSpec((2, _NBLK, H), lambda i: (0, i, 0)),
            pl.BlockSpec((_NBLK, H), lambda i: (i, 0)),
            _full((H, 2 * H)),
            _full((1, 2 * H)),
            _full((1, 2 * H)),
            _full((1, 2 * H)),
            _full((2 * H, H)),
            _full((1, H)),
            _full((1, H)),
            _full((1, H)),
        ],
        out_specs=[
            pl.BlockSpec((_NBLK, H), lambda i: (i, 0)),
            pl.BlockSpec((_NBLK, H), lambda i: (i, 0)),
        ],
        out_shape=[
            jax.ShapeDtypeStruct((N, H), jnp.float32),
            jax.ShapeDtypeStruct((N, H), jnp.float32),
        ],
    )(agg0, h, W1_0, _row(b1_0), _row(g1_0), _row(be1_0), W2_0, _row(b2_0),
      _row(ln_g1), _row(ln_b1))

    agg1 = _sc_aggregate()(r, e2_flat, src2, dst2,
                           jnp.full((16,), t1, jnp.float32))

    out = pl.pallas_call(
        _update2_body,
        grid=(N // _NBLK,),
        in_specs=[
            pl.BlockSpec((2, _NBLK, H), lambda i: (0, i, 0)),
            pl.BlockSpec((_NBLK, H), lambda i: (i, 0)),
            pl.BlockSpec((_NBLK, H), lambda i: (i, 0)),
            _full((H, 2 * H)),
            _full((1, 2 * H)),
            _full((1, 2 * H)),
            _full((1, 2 * H)),
            _full((2 * H, H)),
            _full((1, H)),
            _full((1, H)),
            _full((1, H)),
            _full((H, H)),
            _full((1, H)),
        ],
        out_specs=pl.BlockSpec((_NBLK, H), lambda i: (i, 0)),
        out_shape=jax.ShapeDtypeStruct((N, H), jnp.float32),
    )(agg1, r, h1, W1_1, _row(b1_1), _row(g1_1), _row(be1_1), W2_1,
      _row(b2_1), _row(ln_g0), _row(ln_b0), out_W, _row(out_b))

    return out


# speedup vs baseline: 4.2029x; 1.5439x over previous
import functools

import jax
import jax.numpy as jnp
from jax import lax
from jax.experimental import pallas as pl
from jax.experimental.pallas import tpu as pltpu
from jax.experimental.pallas import tpu_sc as plsc

N = 10000
E = 320000
D_EDGE = 16
H = 128
HH = 64

NUM_TILES = 16
CHUNK = 32
CH2 = CHUNK // 2
NCH = 640
IBLK = 32
NIB = NCH // IBLK
E_PAD = NUM_TILES * NCH * CHUNK
N_ACC = 10240
ROWS_PER_TILE = N_ACC // NUM_TILES

def _sc_aggregate_body(htab, e2, src2, dst2, tvec, agg, acc, sidx, didx, g_v,
                       e_v, o_v, tv, gsem, esem, ssem):
    c = lax.axis_index("c")
    s = lax.axis_index("s")

    pltpu.sync_copy(tvec, tv)

    def _zrow(i, _):
        for b in range(2):
            for k in range(8):
                o_v[b, i, pl.ds(16 * k, 16)] = jnp.zeros((16,), jnp.float32)
        return 0

    lax.fori_loop(0, CHUNK, _zrow, 0)

    zbase = s * ROWS_PER_TILE

    def _zcopy(j, _):
        z0 = pl.multiple_of(zbase + j * CHUNK, 8)
        pltpu.sync_copy(o_v.at[0], acc.at[pl.ds(z0, CHUNK)])
        return 0

    lax.fori_loop(0, ROWS_PER_TILE // CHUNK, _zcopy, 0)

    plsc.subcore_barrier()

    wbase = s * NCH * CHUNK
    irow = s * NCH
    e_base = c * (E_PAD // 2) + wbase // 2
    cb = c * HH
    tvv = tv[...]

    def _fire_in(ci, b):
        q = lax.rem(ci, IBLK)
        s2 = lax.rem(ci // IBLK, 2)
        pltpu.async_copy(htab.at[sidx.at[s2, q]], g_v.at[b], gsem)
        eb = pl.multiple_of(e_base + ci * CH2, 8)
        pltpu.async_copy(e2.at[pl.ds(eb, CH2)], e_v.at[b], esem)

    pltpu.sync_copy(src2.at[pl.ds(pl.multiple_of(irow, 8), IBLK)], sidx.at[0])
    pltpu.sync_copy(dst2.at[pl.ds(pl.multiple_of(irow, 8), IBLK)], didx)
    for b in range(2):
        _fire_in(b, b)

    def _blk(bi, _):
        @pl.when(bi > 0)
        def _():
            for _i in range(2):
                pltpu.make_async_copy(o_v.at[0], acc.at[didx.at[0]], ssem).wait()
            r0 = pl.multiple_of(irow + bi * IBLK, 8)
            pltpu.sync_copy(dst2.at[pl.ds(r0, IBLK)], didx)

        @pl.when(bi < NIB - 1)
        def _():
            r1 = pl.multiple_of(irow + (bi + 1) * IBLK, 8)
            pltpu.sync_copy(src2.at[pl.ds(r1, IBLK)], sidx.at[lax.rem(bi + 1, 2)])

        def _pair(cp, _):
            for b in range(2):
                lq = 2 * cp + b
                ci = bi * IBLK + lq
                pltpu.make_async_copy(htab.at[sidx.at[0, 0]], g_v.at[b], gsem).wait()
                pltpu.make_async_copy(e2.at[pl.ds(0, CH2)], e_v.at[b], esem).wait()

                @pl.when(cp >= 1)
                def _():
                    pltpu.make_async_copy(o_v.at[0], acc.at[didx.at[0]], ssem).wait()

                @plsc.parallel_loop(0, CH2, unroll=2)
                def _edge(rr):
                    for half in range(2):
                        i = 2 * rr + half
                        for k in range(4):
                            g = g_v[b, i, pl.ds(cb + 16 * k, 16)]
                            ee = e_v[b, rr, pl.ds(HH * half + 16 * k, 16)]
                            m = jnp.maximum(g + ee, 0.0) + 1e-7
                            w = jnp.exp(m * tvv)
                            o_v[b, i, pl.ds(16 * k, 16)] = m * w
                            o_v[b, i, pl.ds(HH + 16 * k, 16)] = w
                pltpu.async_copy(o_v.at[b], acc.at[didx.at[lq]], ssem, add=True)

                @pl.when(ci + 2 < NCH)
                def _():
                    _fire_in(ci + 2, b)
            return 0

        lax.fori_loop(0, IBLK // 2, _pair, 0)
        return 0

    lax.fori_loop(0, NIB, _blk, 0)

    for _i in range(2):
        pltpu.make_async_copy(o_v.at[0], acc.at[didx.at[0]], ssem).wait()

    plsc.subcore_barrier()

    rbase = s * ROWS_PER_TILE

    def _out(j, _):
        r0 = pl.multiple_of(rbase + j * CHUNK, 8)
        pltpu.sync_copy(acc.at[pl.ds(r0, CHUNK)], o_v.at[0])

        @plsc.parallel_loop(0, CHUNK, unroll=2)
        def _row(r):
            for k in range(4):
                num = o_v[0, r, pl.ds(16 * k, 16)]
                den = o_v[0, r, pl.ds(HH + 16 * k, 16)]
                g_v[0, r, pl.ds(cb + 16 * k, 16)] = num / (den + 1e-16)
        pltpu.sync_copy(g_v.at[0], agg.at[c, pl.ds(r0, CHUNK)])
        return 0

    lax.fori_loop(0, ROWS_PER_TILE // CHUNK, _out, 0)


@functools.cache
def _sc_aggregate():
    mesh = plsc.VectorSubcoreMesh(
        core_axis_name="c", subcore_axis_name="s", num_cores=2, num_subcores=16
    )
    return pl.kernel(
        _sc_aggregate_body,
        out_type=jax.ShapeDtypeStruct((2, N_ACC, H), jnp.float32),
        mesh=mesh,
        scratch_types=[
            pltpu.VMEM_SHARED((N_ACC, H), jnp.float32),
            pltpu.VMEM((2, IBLK, CHUNK), jnp.int32),
            pltpu.VMEM((IBLK, CHUNK), jnp.int32),
            pltpu.VMEM((2, CHUNK, H), jnp.float32),
            pltpu.VMEM((2, CH2, H), jnp.float32),
            pltpu.VMEM((2, CHUNK, H), jnp.float32),
            pltpu.VMEM((16,), jnp.float32),
            pltpu.SemaphoreType.DMA,
            pltpu.SemaphoreType.DMA,
            pltpu.SemaphoreType.DMA,
        ],
    )


_NBLK = 1000
_EBLK = 2048


def _ln(z, g, b):
    mu = jnp.mean(z, axis=-1, keepdims=True)
    var = jnp.mean((z - mu) ** 2, axis=-1, keepdims=True)
    return (z - mu) / jnp.sqrt(var + 1e-5) * g + b


def _enc_node_body(x_ref, w_ref, b_ref, out_ref):
    h = jnp.dot(x_ref[...], w_ref[...], preferred_element_type=jnp.float32)
    out_ref[...] = h + b_ref[...]


def _enc_edge_body(a_ref, wd_ref, bd_ref, out_ref):
    a = a_ref[...]
    for c in range(2):
        z = jnp.dot(a, wd_ref[c], preferred_element_type=jnp.float32)
        out_ref[c] = jax.nn.sigmoid(z + bd_ref[c])


def _update1_body(agg_ref, h_ref, w1_ref, b1_ref, g1_ref, be1_ref, w2_ref,
                  b2_ref, lng_ref, lnb_ref, h1_ref, r_ref):
    agg = jnp.concatenate([agg_ref[0][:, :HH], agg_ref[1][:, HH:]], axis=1)
    u = h_ref[...] + agg
    z = jnp.dot(u, w1_ref[...], preferred_element_type=jnp.float32) + b1_ref[...]
    z = jax.nn.relu(_ln(z, g1_ref[...], be1_ref[...]))
    h1 = jnp.dot(z, w2_ref[...], preferred_element_type=jnp.float32) + b2_ref[...]
    h1_ref[...] = h1
    r_ref[...] = jax.nn.relu(_ln(h1, lng_ref[...], lnb_ref[...]))


def _update2_body(agg_ref, r_ref, h1_ref, w1_ref, b1_ref, g1_ref, be1_ref,
                  w2_ref, b2_ref, lng_ref, lnb_ref, ow_ref, ob_ref, out_ref):
    agg = jnp.concatenate([agg_ref[0][:, :HH], agg_ref[1][:, HH:]], axis=1)
    u = r_ref[...] + agg
    z = jnp.dot(u, w1_ref[...], preferred_element_type=jnp.float32) + b1_ref[...]
    z = jax.nn.relu(_ln(z, g1_ref[...], be1_ref[...]))
    h2 = h1_ref[...] + jnp.dot(z, w2_ref[...], preferred_element_type=jnp.float32) + b2_ref[...]
    f = jax.nn.relu(_ln(h2, lng_ref[...], lnb_ref[...]))
    out_ref[...] = jnp.dot(f, ow_ref[...], preferred_element_type=jnp.float32) + ob_ref[...]


def _full(shape):
    return pl.BlockSpec(shape, lambda i: tuple(0 for _ in shape))


def _row(v):
    return v.reshape(1, -1)


def kernel(x, edge_index, edge_attr, node_W, node_b, edge_W, edge_b, t0, W1_0,
           b1_0, g1_0, be1_0, W2_0, b2_0, ln_g0, ln_b0, t1, W1_1, b1_1, g1_1,
           be1_1, W2_1, b2_1, ln_g1, ln_b1, out_W, out_b):
    pad = E_PAD - E
    src_p = jnp.concatenate([edge_index[0], jnp.zeros((pad,), jnp.int32)])
    dst_p = jnp.concatenate([edge_index[1], jnp.full((pad,), N, jnp.int32)])
    ea_p = jnp.concatenate([edge_attr, jnp.zeros((pad, D_EDGE), jnp.float32)])

    h = pl.pallas_call(
        _enc_node_body,
        grid=(N // _NBLK,),
        in_specs=[
            pl.BlockSpec((_NBLK, H), lambda i: (i, 0)),
            _full((H, H)),
            _full((1, H)),
        ],
        out_specs=pl.BlockSpec((_NBLK, H), lambda i: (i, 0)),
        out_shape=jax.ShapeDtypeStruct((N, H), jnp.float32),
    )(x, node_W, _row(node_b))

    ea2 = ea_p.reshape(E_PAD // 2, 2 * D_EDGE)
    z16 = jnp.zeros((D_EDGE, HH), jnp.float32)
    wd = jnp.stack([
        jnp.concatenate([
            jnp.concatenate([edge_W[:, c * HH:(c + 1) * HH], z16], axis=1),
            jnp.concatenate([z16, edge_W[:, c * HH:(c + 1) * HH]], axis=1),
        ], axis=0) for c in range(2)
    ])
    bd = jnp.stack([jnp.tile(edge_b[c * HH:(c + 1) * HH], 2) for c in range(2)])

    e2 = pl.pallas_call(
        _enc_edge_body,
        grid=(E_PAD // _EBLK,),
        in_specs=[
            pl.BlockSpec((_EBLK // 2, 2 * D_EDGE), lambda i: (i, 0)),
            _full((2, 2 * D_EDGE, H)),
            _full((2, H)),
        ],
        out_specs=pl.BlockSpec((2, _EBLK // 2, H), lambda i: (0, i, 0)),
        out_shape=jax.ShapeDtypeStruct((2, E_PAD // 2, H), jnp.float32),
    )(ea2, wd, bd)

    e2_flat = e2.reshape(E_PAD, H)

    src2 = src_p.reshape(E_PAD // CHUNK, CHUNK)
    dst2 = dst_p.reshape(E_PAD // CHUNK, CHUNK)

    agg0 = _sc_aggregate()(h, e2_flat, src2, dst2,
                           jnp.full((16,), t0, jnp.float32))

    h1, r = pl.pallas_call(
        _update1_body,
        grid=(N // _NBLK,),
        in_specs=[
            pl.BlockSpec((2, _NBLK, H), lambda i: (0, i, 0)),
            pl.BlockSpec((_NBLK, H), lambda i: (i, 0)),
            _full((H, 2 * H)),
            _full((1, 2 * H)),
            _full((1, 2 * H)),
            _full((1, 2 * H)),
            _full((2 * H, H)),
            _full((1, H)),
            _full((1, H)),
            _full((1, H)),
        ],
        out_specs=[
            pl.BlockSpec((_NBLK, H), lambda i: (i, 0)),
            pl.BlockSpec((_NBLK, H), lambda i: (i, 0)),
        ],
        out_shape=[
            jax.ShapeDtypeStruct((N, H), jnp.float32),
            jax.ShapeDtypeStruct((N, H), jnp.float32),
        ],
    )(agg0, h, W1_0, _row(b1_0), _row(g1_0), _row(be1_0), W2_0, _row(b2_0),
      _row(ln_g1), _row(ln_b1))

    agg1 = _sc_aggregate()(r, e2_flat, src2, dst2,
                           jnp.full((16,), t1, jnp.float32))

    out = pl.pallas_call(
        _update2_body,
        grid=(N // _NBLK,),
        in_specs=[
            pl.BlockSpec((2, _NBLK, H), lambda i: (0, i, 0)),
            pl.BlockSpec((_NBLK, H), lambda i: (i, 0)),
            pl.BlockSpec((_NBLK, H), lambda i: (i, 0)),
            _full((H, 2 * H)),
            _full((1, 2 * H)),
            _full((1, 2 * H)),
            _full((1, 2 * H)),
            _full((2 * H, H)),
            _full((1, H)),
            _full((1, H)),
            _full((1, H)),
            _full((H, H)),
            _full((1, H)),
        ],
        out_specs=pl.BlockSpec((_NBLK, H), lambda i: (i, 0)),
        out_shape=jax.ShapeDtypeStruct((N, H), jnp.float32),
    )(agg1, r, h1, W1_1, _row(b1_1), _row(g1_1), _row(be1_1), W2_1,
      _row(b2_1), _row(ln_g0), _row(ln_b0), out_W, _row(out_b))

    return out


# speedup vs baseline: 4.3734x; 1.0406x over previous
import functools

import jax
import jax.numpy as jnp
from jax import lax
from jax.experimental import pallas as pl
from jax.experimental.pallas import tpu as pltpu
from jax.experimental.pallas import tpu_sc as plsc

N = 10000
E = 320000
D_EDGE = 16
H = 128
HH = 64

NUM_TILES = 16
CHUNK = 64
CH2 = CHUNK // 2
NCH = 320
IBLK = 8
NIB = NCH // IBLK
E_PAD = NUM_TILES * NCH * CHUNK
N_ACC = 10240
ROWS_PER_TILE = N_ACC // NUM_TILES

def _sc_aggregate_body(htab, e2, src2, dst2, tvec, agg, acc, sidx, didx, g_v,
                       e_v, o_v, tv, gsem, esem, ssem):
    c = lax.axis_index("c")
    s = lax.axis_index("s")

    pltpu.sync_copy(tvec, tv)

    def _zrow(i, _):
        for b in range(2):
            for k in range(8):
                o_v[b, i, pl.ds(16 * k, 16)] = jnp.zeros((16,), jnp.float32)
        return 0

    lax.fori_loop(0, CHUNK, _zrow, 0)

    zbase = s * ROWS_PER_TILE

    def _zcopy(j, _):
        z0 = pl.multiple_of(zbase + j * CHUNK, 8)
        pltpu.sync_copy(o_v.at[0], acc.at[pl.ds(z0, CHUNK)])
        return 0

    lax.fori_loop(0, ROWS_PER_TILE // CHUNK, _zcopy, 0)

    plsc.subcore_barrier()

    wbase = s * NCH * CHUNK
    irow = s * NCH
    e_base = c * (E_PAD // 2) + wbase // 2
    cb = c * HH
    tvv = tv[...]

    def _fire_in(ci, b):
        q = lax.rem(ci, IBLK)
        s2 = lax.rem(ci // IBLK, 2)
        pltpu.async_copy(htab.at[sidx.at[s2, q]], g_v.at[b], gsem)
        eb = pl.multiple_of(e_base + ci * CH2, 8)
        pltpu.async_copy(e2.at[pl.ds(eb, CH2)], e_v.at[b], esem)

    pltpu.sync_copy(src2.at[pl.ds(pl.multiple_of(irow, 8), IBLK)], sidx.at[0])
    pltpu.sync_copy(dst2.at[pl.ds(pl.multiple_of(irow, 8), IBLK)], didx)
    for b in range(2):
        _fire_in(b, b)

    def _blk(bi, _):
        @pl.when(bi > 0)
        def _():
            for _i in range(2):
                pltpu.make_async_copy(o_v.at[0], acc.at[didx.at[0]], ssem).wait()
            r0 = pl.multiple_of(irow + bi * IBLK, 8)
            pltpu.sync_copy(dst2.at[pl.ds(r0, IBLK)], didx)

        @pl.when(bi < NIB - 1)
        def _():
            r1 = pl.multiple_of(irow + (bi + 1) * IBLK, 8)
            pltpu.sync_copy(src2.at[pl.ds(r1, IBLK)], sidx.at[lax.rem(bi + 1, 2)])

        def _pair(cp, _):
            for b in range(2):
                lq = 2 * cp + b
                ci = bi * IBLK + lq
                pltpu.make_async_copy(htab.at[sidx.at[0, 0]], g_v.at[b], gsem).wait()
                pltpu.make_async_copy(e2.at[pl.ds(0, CH2)], e_v.at[b], esem).wait()

                @pl.when(cp >= 1)
                def _():
                    pltpu.make_async_copy(o_v.at[0], acc.at[didx.at[0]], ssem).wait()

                @plsc.parallel_loop(0, CH2, unroll=2)
                def _edge(rr):
                    for half in range(2):
                        i = 2 * rr + half
                        for k in range(4):
                            g = g_v[b, i, pl.ds(cb + 16 * k, 16)]
                            ee = e_v[b, rr, pl.ds(HH * half + 16 * k, 16)]
                            m = jnp.maximum(g + ee, 0.0) + 1e-7
                            w = jnp.exp(m * tvv)
                            o_v[b, i, pl.ds(16 * k, 16)] = m * w
                            o_v[b, i, pl.ds(HH + 16 * k, 16)] = w
                pltpu.async_copy(o_v.at[b], acc.at[didx.at[lq]], ssem, add=True)

                @pl.when(ci + 2 < NCH)
                def _():
                    _fire_in(ci + 2, b)
            return 0

        lax.fori_loop(0, IBLK // 2, _pair, 0)
        return 0

    lax.fori_loop(0, NIB, _blk, 0)

    for _i in range(2):
        pltpu.make_async_copy(o_v.at[0], acc.at[didx.at[0]], ssem).wait()

    plsc.subcore_barrier()

    rbase = s * ROWS_PER_TILE

    def _out(j, _):
        r0 = pl.multiple_of(rbase + j * CHUNK, 8)
        pltpu.sync_copy(acc.at[pl.ds(r0, CHUNK)], o_v.at[0])

        @plsc.parallel_loop(0, CHUNK, unroll=2)
        def _row(r):
            for k in range(4):
                num = o_v[0, r, pl.ds(16 * k, 16)]
                den = o_v[0, r, pl.ds(HH + 16 * k, 16)]
                g_v[0, r, pl.ds(cb + 16 * k, 16)] = num / (den + 1e-16)
        pltpu.sync_copy(g_v.at[0], agg.at[c, pl.ds(r0, CHUNK)])
        return 0

    lax.fori_loop(0, ROWS_PER_TILE // CHUNK, _out, 0)


@functools.cache
def _sc_aggregate():
    mesh = plsc.VectorSubcoreMesh(
        core_axis_name="c", subcore_axis_name="s", num_cores=2, num_subcores=16
    )
    return pl.kernel(
        _sc_aggregate_body,
        out_type=jax.ShapeDtypeStruct((2, N_ACC, H), jnp.float32),
        mesh=mesh,
        scratch_types=[
            pltpu.VMEM_SHARED((N_ACC, H), jnp.float32),
            pltpu.VMEM((2, IBLK, CHUNK), jnp.int32),
            pltpu.VMEM((IBLK, CHUNK), jnp.int32),
            pltpu.VMEM((2, CHUNK, H), jnp.float32),
            pltpu.VMEM((2, CH2, H), jnp.float32),
            pltpu.VMEM((2, CHUNK, H), jnp.float32),
            pltpu.VMEM((16,), jnp.float32),
            pltpu.SemaphoreType.DMA,
            pltpu.SemaphoreType.DMA,
            pltpu.SemaphoreType.DMA,
        ],
    )


_NBLK = 1000
_EBLK = 2048


def _ln(z, g, b):
    mu = jnp.mean(z, axis=-1, keepdims=True)
    var = jnp.mean((z - mu) ** 2, axis=-1, keepdims=True)
    return (z - mu) / jnp.sqrt(var + 1e-5) * g + b


def _enc_node_body(x_ref, w_ref, b_ref, out_ref):
    h = jnp.dot(x_ref[...], w_ref[...], preferred_element_type=jnp.float32)
    out_ref[...] = h + b_ref[...]


def _enc_edge_body(a_ref, wd_ref, bd_ref, out_ref):
    a = a_ref[...]
    for c in range(2):
        z = jnp.dot(a, wd_ref[c], preferred_element_type=jnp.float32)
        out_ref[c] = jax.nn.sigmoid(z + bd_ref[c])


def _update1_body(agg_ref, h_ref, w1_ref, b1_ref, g1_ref, be1_ref, w2_ref,
                  b2_ref, lng_ref, lnb_ref, h1_ref, r_ref):
    agg = jnp.concatenate([agg_ref[0][:, :HH], agg_ref[1][:, HH:]], axis=1)
    u = h_ref[...] + agg
    z = jnp.dot(u, w1_ref[...], preferred_element_type=jnp.float32) + b1_ref[...]
    z = jax.nn.relu(_ln(z, g1_ref[...], be1_ref[...]))
    h1 = jnp.dot(z, w2_ref[...], preferred_element_type=jnp.float32) + b2_ref[...]
    h1_ref[...] = h1
    r_ref[...] = jax.nn.relu(_ln(h1, lng_ref[...], lnb_ref[...]))


def _update2_body(agg_ref, r_ref, h1_ref, w1_ref, b1_ref, g1_ref, be1_ref,
                  w2_ref, b2_ref, lng_ref, lnb_ref, ow_ref, ob_ref, out_ref):
    agg = jnp.concatenate([agg_ref[0][:, :HH], agg_ref[1][:, HH:]], axis=1)
    u = r_ref[...] + agg
    z = jnp.dot(u, w1_ref[...], preferred_element_type=jnp.float32) + b1_ref[...]
    z = jax.nn.relu(_ln(z, g1_ref[...], be1_ref[...]))
    h2 = h1_ref[...] + jnp.dot(z, w2_ref[...], preferred_element_type=jnp.float32) + b2_ref[...]
    f = jax.nn.relu(_ln(h2, lng_ref[...], lnb_ref[...]))
    out_ref[...] = jnp.dot(f, ow_ref[...], preferred_element_type=jnp.float32) + ob_ref[...]


def _full(shape):
    return pl.BlockSpec(shape, lambda i: tuple(0 for _ in shape))


def _row(v):
    return v.reshape(1, -1)


def kernel(x, edge_index, edge_attr, node_W, node_b, edge_W, edge_b, t0, W1_0,
           b1_0, g1_0, be1_0, W2_0, b2_0, ln_g0, ln_b0, t1, W1_1, b1_1, g1_1,
           be1_1, W2_1, b2_1, ln_g1, ln_b1, out_W, out_b):
    pad = E_PAD - E
    src_p = jnp.concatenate([edge_index[0], jnp.zeros((pad,), jnp.int32)])
    dst_p = jnp.concatenate([edge_index[1], jnp.full((pad,), N, jnp.int32)])
    ea_p = jnp.concatenate([edge_attr, jnp.zeros((pad, D_EDGE), jnp.float32)])

    h = pl.pallas_call(
        _enc_node_body,
        grid=(N // _NBLK,),
        in_specs=[
            pl.BlockSpec((_NBLK, H), lambda i: (i, 0)),
            _full((H, H)),
            _full((1, H)),
        ],
        out_specs=pl.BlockSpec((_NBLK, H), lambda i: (i, 0)),
        out_shape=jax.ShapeDtypeStruct((N, H), jnp.float32),
    )(x, node_W, _row(node_b))

    ea2 = ea_p.reshape(E_PAD // 2, 2 * D_EDGE)
    z16 = jnp.zeros((D_EDGE, HH), jnp.float32)
    wd = jnp.stack([
        jnp.concatenate([
            jnp.concatenate([edge_W[:, c * HH:(c + 1) * HH], z16], axis=1),
            jnp.concatenate([z16, edge_W[:, c * HH:(c + 1) * HH]], axis=1),
        ], axis=0) for c in range(2)
    ])
    bd = jnp.stack([jnp.tile(edge_b[c * HH:(c + 1) * HH], 2) for c in range(2)])

    e2 = pl.pallas_call(
        _enc_edge_body,
        grid=(E_PAD // _EBLK,),
        in_specs=[
            pl.BlockSpec((_EBLK // 2, 2 * D_EDGE), lambda i: (i, 0)),
            _full((2, 2 * D_EDGE, H)),
            _full((2, H)),
        ],
        out_specs=pl.BlockSpec((2, _EBLK // 2, H), lambda i: (0, i, 0)),
        out_shape=jax.ShapeDtypeStruct((2, E_PAD // 2, H), jnp.float32),
    )(ea2, wd, bd)

    e2_flat = e2.reshape(E_PAD, H)

    src2 = src_p.reshape(E_PAD // CHUNK, CHUNK)
    dst2 = dst_p.reshape(E_PAD // CHUNK, CHUNK)

    agg0 = _sc_aggregate()(h, e2_flat, src2, dst2,
                           jnp.full((16,), t0, jnp.float32))

    h1, r = pl.pallas_call(
        _update1_body,
        grid=(N // _NBLK,),
        in_specs=[
            pl.BlockSpec((2, _NBLK, H), lambda i: (0, i, 0)),
            pl.BlockSpec((_NBLK, H), lambda i: (i, 0)),
            _full((H, 2 * H)),
            _full((1, 2 * H)),
            _full((1, 2 * H)),
            _full((1, 2 * H)),
            _full((2 * H, H)),
            _full((1, H)),
            _full((1, H)),
            _full((1, H)),
        ],
        out_specs=[
            pl.BlockSpec((_NBLK, H), lambda i: (i, 0)),
            pl.BlockSpec((_NBLK, H), lambda i: (i, 0)),
        ],
        out_shape=[
            jax.ShapeDtypeStruct((N, H), jnp.float32),
            jax.ShapeDtypeStruct((N, H), jnp.float32),
        ],
    )(agg0, h, W1_0, _row(b1_0), _row(g1_0), _row(be1_0), W2_0, _row(b2_0),
      _row(ln_g1), _row(ln_b1))

    agg1 = _sc_aggregate()(r, e2_flat, src2, dst2,
                           jnp.full((16,), t1, jnp.float32))

    out = pl.pallas_call(
        _update2_body,
        grid=(N // _NBLK,),
        in_specs=[
            pl.BlockSpec((2, _NBLK, H), lambda i: (0, i, 0)),
            pl.BlockSpec((_NBLK, H), lambda i: (i, 0)),
            pl.BlockSpec((_NBLK, H), lambda i: (i, 0)),
            _full((H, 2 * H)),
            _full((1, 2 * H)),
            _full((1, 2 * H)),
            _full((1, 2 * H)),
            _full((2 * H, H)),
            _full((1, H)),
            _full((1, H)),
            _full((1, H)),
            _full((H, H)),
            _full((1, H)),
        ],
        out_specs=pl.BlockSpec((_NBLK, H), lambda i: (i, 0)),
        out_shape=jax.ShapeDtypeStruct((N, H), jnp.float32),
    )(agg1, r, h1, W1_1, _row(b1_1), _row(g1_1), _row(be1_1), W2_1,
      _row(b2_1), _row(ln_g0), _row(ln_b0), out_W, _row(out_b))

    return out


# speedup vs baseline: 6.3169x; 1.4444x over previous
import functools

import jax
import jax.numpy as jnp
from jax import lax
from jax.experimental import pallas as pl
from jax.experimental.pallas import tpu as pltpu
from jax.experimental.pallas import tpu_sc as plsc

N = 10000
E = 320000
D_EDGE = 16
H = 128
HH = 64

NUM_TILES = 16
CHUNK = 64
CH2 = CHUNK // 2
NCH = 320
IBLK = 8
NIB = NCH // IBLK
E_PAD = NUM_TILES * NCH * CHUNK
N_ACC = 10240
ROWS_PER_TILE = N_ACC // NUM_TILES

def _sc_aggregate_body(htab, e2, src2, dst2, tvec, agg, acc, sidx, didx, g_v,
                       e_v, o_v, tv, gsem, esem, ssem):
    c = lax.axis_index("c")
    s = lax.axis_index("s")

    pltpu.sync_copy(tvec, tv)

    def _zrow(i, _):
        for b in range(2):
            for k in range(8):
                o_v[b, i, pl.ds(16 * k, 16)] = jnp.zeros((16,), jnp.float32)
        return 0

    lax.fori_loop(0, CHUNK, _zrow, 0)

    zbase = s * ROWS_PER_TILE

    def _zcopy(j, _):
        z0 = pl.multiple_of(zbase + j * CHUNK, 8)
        pltpu.sync_copy(o_v.at[0], acc.at[pl.ds(z0, CHUNK)])
        return 0

    lax.fori_loop(0, ROWS_PER_TILE // CHUNK, _zcopy, 0)

    plsc.subcore_barrier()

    wbase = s * NCH * CHUNK
    irow = s * NCH
    e_base = c * E_PAD + wbase
    cb = c * HH
    tvv = tv[...]

    def _fire_in(ci, b):
        q = lax.rem(ci, IBLK)
        s2 = lax.rem(ci // IBLK, 2)
        pltpu.async_copy(htab.at[sidx.at[s2, q]], g_v.at[b], gsem)
        eb = pl.multiple_of(e_base + ci * CHUNK, 8)
        pltpu.async_copy(e2.at[pl.ds(eb, CHUNK)], e_v.at[b], esem)

    pltpu.sync_copy(src2.at[c, pl.ds(pl.multiple_of(irow, 8), IBLK)],
                    sidx.at[0])
    pltpu.sync_copy(dst2.at[pl.ds(pl.multiple_of(irow, 8), IBLK)], didx)
    for b in range(2):
        _fire_in(b, b)

    def _blk(bi, _):
        @pl.when(bi > 0)
        def _():
            for _i in range(2):
                pltpu.make_async_copy(o_v.at[0], acc.at[didx.at[0]], ssem).wait()
            r0 = pl.multiple_of(irow + bi * IBLK, 8)
            pltpu.sync_copy(dst2.at[pl.ds(r0, IBLK)], didx)

        @pl.when(bi < NIB - 1)
        def _():
            r1 = pl.multiple_of(irow + (bi + 1) * IBLK, 8)
            pltpu.sync_copy(src2.at[c, pl.ds(r1, IBLK)],
                            sidx.at[lax.rem(bi + 1, 2)])

        def _pair(cp, _):
            for b in range(2):
                lq = 2 * cp + b
                ci = bi * IBLK + lq
                pltpu.make_async_copy(htab.at[sidx.at[0, 0]], g_v.at[b], gsem).wait()
                pltpu.make_async_copy(e2.at[pl.ds(0, CHUNK)], e_v.at[b], esem).wait()

                @pl.when(cp >= 1)
                def _():
                    pltpu.make_async_copy(o_v.at[0], acc.at[didx.at[0]], ssem).wait()

                @plsc.parallel_loop(0, CHUNK, unroll=2)
                def _edge(i):
                    for k in range(4):
                        g = g_v[b, i, pl.ds(16 * k, 16)]
                        ee = e_v[b, i, pl.ds(16 * k, 16)]
                        m = jnp.maximum(g + ee, 0.0) + 1e-7
                        w = jnp.exp(m * tvv)
                        o_v[b, i, pl.ds(16 * k, 16)] = m * w
                        o_v[b, i, pl.ds(HH + 16 * k, 16)] = w
                pltpu.async_copy(o_v.at[b], acc.at[didx.at[lq]], ssem, add=True)

                @pl.when(ci + 2 < NCH)
                def _():
                    _fire_in(ci + 2, b)
            return 0

        lax.fori_loop(0, IBLK // 2, _pair, 0)
        return 0

    lax.fori_loop(0, NIB, _blk, 0)

    for _i in range(2):
        pltpu.make_async_copy(o_v.at[0], acc.at[didx.at[0]], ssem).wait()

    plsc.subcore_barrier()

    rbase = s * ROWS_PER_TILE

    def _out(j, _):
        r0 = pl.multiple_of(rbase + j * CHUNK, 8)
        pltpu.sync_copy(acc.at[pl.ds(r0, CHUNK)], o_v.at[0])

        @plsc.parallel_loop(0, CHUNK, unroll=2)
        def _row(r):
            for k in range(4):
                num = o_v[0, r, pl.ds(16 * k, 16)]
                den = o_v[0, r, pl.ds(HH + 16 * k, 16)]
                e_v[0, r, pl.ds(16 * k, 16)] = num / (den + 1e-16)
        pltpu.sync_copy(e_v.at[0], agg.at[c, pl.ds(r0, CHUNK)])
        return 0

    lax.fori_loop(0, ROWS_PER_TILE // CHUNK, _out, 0)


@functools.cache
def _sc_aggregate():
    mesh = plsc.VectorSubcoreMesh(
        core_axis_name="c", subcore_axis_name="s", num_cores=2, num_subcores=16
    )
    return pl.kernel(
        _sc_aggregate_body,
        out_type=jax.ShapeDtypeStruct((2, N_ACC, HH), jnp.float32),
        mesh=mesh,
        compiler_params=pltpu.CompilerParams(use_tc_tiling_on_sc=False),
        scratch_types=[
            pltpu.VMEM_SHARED((N_ACC, H), jnp.float32),
            pltpu.VMEM((2, IBLK, CHUNK), jnp.int32),
            pltpu.VMEM((IBLK, CHUNK), jnp.int32),
            pltpu.VMEM((2, CHUNK, HH), jnp.float32),
            pltpu.VMEM((2, CHUNK, HH), jnp.float32),
            pltpu.VMEM((2, CHUNK, H), jnp.float32),
            pltpu.VMEM((16,), jnp.float32),
            pltpu.SemaphoreType.DMA,
            pltpu.SemaphoreType.DMA,
            pltpu.SemaphoreType.DMA,
        ],
    )


_NBLK = 1000
_EBLK = 2048


def _ln(z, g, b):
    mu = jnp.mean(z, axis=-1, keepdims=True)
    var = jnp.mean((z - mu) ** 2, axis=-1, keepdims=True)
    return (z - mu) / jnp.sqrt(var + 1e-5) * g + b


def _enc_node_body(x_ref, w_ref, b_ref, h_ref, hp_ref):
    h = jnp.dot(x_ref[...], w_ref[...], preferred_element_type=jnp.float32)
    h = h + b_ref[...]
    h_ref[...] = h
    hp_ref[...] = jnp.stack([h[:, :HH], h[:, HH:]])


def _enc_edge_body(a_ref, w_ref, b_ref, out_ref):
    z = jnp.dot(a_ref[...], w_ref[...], preferred_element_type=jnp.float32)
    z = jax.nn.sigmoid(z + b_ref[...])
    out_ref[...] = jnp.stack([z[:, :HH], z[:, HH:]])


def _update1_body(agg_ref, h_ref, w1_ref, b1_ref, g1_ref, be1_ref, w2_ref,
                  b2_ref, lng_ref, lnb_ref, h1_ref, r_ref, rp_ref):
    agg = jnp.concatenate([agg_ref[0], agg_ref[1]], axis=1)
    u = h_ref[...] + agg
    z = jnp.dot(u, w1_ref[...], preferred_element_type=jnp.float32) + b1_ref[...]
    z = jax.nn.relu(_ln(z, g1_ref[...], be1_ref[...]))
    h1 = jnp.dot(z, w2_ref[...], preferred_element_type=jnp.float32) + b2_ref[...]
    h1_ref[...] = h1
    r = jax.nn.relu(_ln(h1, lng_ref[...], lnb_ref[...]))
    r_ref[...] = r
    rp_ref[...] = jnp.stack([r[:, :HH], r[:, HH:]])


def _update2_body(agg_ref, r_ref, h1_ref, w1_ref, b1_ref, g1_ref, be1_ref,
                  w2_ref, b2_ref, lng_ref, lnb_ref, ow_ref, ob_ref, out_ref):
    agg = jnp.concatenate([agg_ref[0], agg_ref[1]], axis=1)
    u = r_ref[...] + agg
    z = jnp.dot(u, w1_ref[...], preferred_element_type=jnp.float32) + b1_ref[...]
    z = jax.nn.relu(_ln(z, g1_ref[...], be1_ref[...]))
    h2 = h1_ref[...] + jnp.dot(z, w2_ref[...], preferred_element_type=jnp.float32) + b2_ref[...]
    f = jax.nn.relu(_ln(h2, lng_ref[...], lnb_ref[...]))
    out_ref[...] = jnp.dot(f, ow_ref[...], preferred_element_type=jnp.float32) + ob_ref[...]


def _full(shape):
    return pl.BlockSpec(shape, lambda i: tuple(0 for _ in shape))


def _row(v):
    return v.reshape(1, -1)


def kernel(x, edge_index, edge_attr, node_W, node_b, edge_W, edge_b, t0, W1_0,
           b1_0, g1_0, be1_0, W2_0, b2_0, ln_g0, ln_b0, t1, W1_1, b1_1, g1_1,
           be1_1, W2_1, b2_1, ln_g1, ln_b1, out_W, out_b):
    pad = E_PAD - E
    src_p = jnp.concatenate([edge_index[0], jnp.zeros((pad,), jnp.int32)])
    dst_p = jnp.concatenate([edge_index[1], jnp.full((pad,), N, jnp.int32)])
    ea_p = jnp.concatenate([edge_attr, jnp.zeros((pad, D_EDGE), jnp.float32)])

    h, hp = pl.pallas_call(
        _enc_node_body,
        grid=(N // _NBLK,),
        in_specs=[
            pl.BlockSpec((_NBLK, H), lambda i: (i, 0)),
            _full((H, H)),
            _full((1, H)),
        ],
        out_specs=[
            pl.BlockSpec((_NBLK, H), lambda i: (i, 0)),
            pl.BlockSpec((2, _NBLK, HH), lambda i: (0, i, 0)),
        ],
        out_shape=[
            jax.ShapeDtypeStruct((N, H), jnp.float32),
            jax.ShapeDtypeStruct((2, N, HH), jnp.float32),
        ],
    )(x, node_W, _row(node_b))

    e2 = pl.pallas_call(
        _enc_edge_body,
        grid=(E_PAD // _EBLK,),
        in_specs=[
            pl.BlockSpec((_EBLK, D_EDGE), lambda i: (i, 0)),
            _full((D_EDGE, H)),
            _full((1, H)),
        ],
        out_specs=pl.BlockSpec((2, _EBLK, HH), lambda i: (0, i, 0)),
        out_shape=jax.ShapeDtypeStruct((2, E_PAD, HH), jnp.float32),
    )(ea_p, edge_W, _row(edge_b))

    e2_flat = e2.reshape(2 * E_PAD, HH)

    src2 = src_p.reshape(E_PAD // CHUNK, CHUNK)
    dst2 = dst_p.reshape(E_PAD // CHUNK, CHUNK)
    src3 = jnp.stack([src2, src2 + N])

    agg0 = _sc_aggregate()(hp.reshape(2 * N, HH), e2_flat, src3, dst2,
                           jnp.full((16,), t0, jnp.float32))

    h1, r, rp = pl.pallas_call(
        _update1_body,
        grid=(N // _NBLK,),
        in_specs=[
            pl.BlockSpec((2, _NBLK, HH), lambda i: (0, i, 0)),
            pl.BlockSpec((_NBLK, H), lambda i: (i, 0)),
            _full((H, 2 * H)),
            _full((1, 2 * H)),
            _full((1, 2 * H)),
            _full((1, 2 * H)),
            _full((2 * H, H)),
            _full((1, H)),
            _full((1, H)),
            _full((1, H)),
        ],
        out_specs=[
            pl.BlockSpec((_NBLK, H), lambda i: (i, 0)),
            pl.BlockSpec((_NBLK, H), lambda i: (i, 0)),
            pl.BlockSpec((2, _NBLK, HH), lambda i: (0, i, 0)),
        ],
        out_shape=[
            jax.ShapeDtypeStruct((N, H), jnp.float32),
            jax.ShapeDtypeStruct((N, H), jnp.float32),
            jax.ShapeDtypeStruct((2, N, HH), jnp.float32),
        ],
    )(agg0, h, W1_0, _row(b1_0), _row(g1_0), _row(be1_0), W2_0, _row(b2_0),
      _row(ln_g1), _row(ln_b1))

    agg1 = _sc_aggregate()(rp.reshape(2 * N, HH), e2_flat, src3, dst2,
                           jnp.full((16,), t1, jnp.float32))

    out = pl.pallas_call(
        _update2_body,
        grid=(N // _NBLK,),
        in_specs=[
            pl.BlockSpec((2, _NBLK, HH), lambda i: (0, i, 0)),
            pl.BlockSpec((_NBLK, H), lambda i: (i, 0)),
            pl.BlockSpec((_NBLK, H), lambda i: (i, 0)),
            _full((H, 2 * H)),
            _full((1, 2 * H)),
            _full((1, 2 * H)),
            _full((1, 2 * H)),
            _full((2 * H, H)),
            _full((1, H)),
            _full((1, H)),
            _full((1, H)),
            _full((H, H)),
            _full((1, H)),
        ],
        out_specs=pl.BlockSpec((_NBLK, H), lambda i: (i, 0)),
        out_shape=jax.ShapeDtypeStruct((N, H), jnp.float32),
    )(agg1, r, h1, W1_1, _row(b1_1), _row(g1_1), _row(be1_1), W2_1,
      _row(b2_1), _row(ln_g0), _row(ln_b0), out_W, _row(out_b))

    return out
